# baseline (speedup 1.0000x reference)
import functools

import jax
import jax.numpy as jnp
from jax import lax
from jax.experimental import pallas as pl
from jax.experimental.pallas import tpu as pltpu
from jax.experimental.pallas import tpu_sc as plsc

_LANE = 16


def _make_emb(B, V, D, LINES):
    info = plsc.get_sparse_core_info()
    nw = info.num_cores * info.num_subcores
    assert B % nw == 0
    b_per_w = B // nw
    rpl = 128 // D
    rpl_sh = rpl.bit_length() - 1
    assert (1 << rpl_sh) == rpl and (D & (D - 1)) == 0
    l_per_w = b_per_w // rpl
    chunk = 128 if b_per_w % 128 == 0 else b_per_w
    n_chunks = b_per_w // chunk

    mesh = plsc.VectorSubcoreMesh(core_axis_name="c", subcore_axis_name="s")

    @functools.partial(
        pl.kernel,
        mesh=mesh,
        out_type=jax.ShapeDtypeStruct((B // rpl, 128), jnp.float32),
        scratch_types=[
            pltpu.VMEM((b_per_w,), jnp.int32),
            pltpu.VMEM((b_per_w,), jnp.int32),
            pltpu.VMEM((b_per_w,), jnp.int32),
            pltpu.VMEM((b_per_w,), jnp.int32),
            pltpu.VMEM((b_per_w,), jnp.int32),
            pltpu.VMEM((b_per_w,), jnp.int32),
            pltpu.VMEM((b_per_w, 128), jnp.float32),
            pltpu.VMEM((l_per_w, 128), jnp.float32),
            pltpu.SemaphoreType.DMA,
        ],
        compiler_params=pltpu.CompilerParams(
            use_tc_tiling_on_sc=True,
            needs_layout_passes=False,
        ),
    )
    def emb(
        idx_hbm, lines_hbm, out_hbm,
        idx_v, lid_v, off_v, orow_v, ocol_v, cid_v, line_buf, out_buf, sem,
    ):
        wid = lax.axis_index("s") * info.num_cores + lax.axis_index("c")
        base = wid * b_per_w
        iota16 = lax.iota(jnp.int32, _LANE)
        pltpu.sync_copy(idx_hbm.at[pl.ds(base, b_per_w)], idx_v)
        for c0 in range(0, b_per_w, _LANE):
            v = idx_v[pl.ds(c0, _LANE)]
            lid_v[pl.ds(c0, _LANE)] = lax.shift_right_logical(v, jnp.int32(rpl_sh))
            off_v[pl.ds(c0, _LANE)] = lax.shift_left(
                lax.bitwise_and(v, jnp.int32(rpl - 1)), jnp.int32(D.bit_length() - 1)
            )
            cid = iota16 + jnp.int32(c0)
            cid_v[pl.ds(c0, _LANE)] = cid
            orow_v[pl.ds(c0, _LANE)] = lax.shift_right_logical(cid, jnp.int32(rpl_sh))
            ocol_v[pl.ds(c0, _LANE)] = lax.shift_left(
                lax.bitwise_and(cid, jnp.int32(rpl - 1)),
                jnp.int32(D.bit_length() - 1),
            )
        copies = []
        for j in range(n_chunks):
            copies.append(
                pltpu.async_copy(
                    lines_hbm.at[lid_v.at[pl.ds(j * chunk, chunk)]],
                    line_buf.at[pl.ds(j * chunk, chunk)],
                    sem,
                )
            )
        for c in copies:
            c.wait()

        def grp(g, carry):
            c0 = g * _LANE
            rows_in = cid_v[pl.ds(c0, _LANE)]
            offs = off_v[pl.ds(c0, _LANE)]
            rows_out = orow_v[pl.ds(c0, _LANE)]
            cols_base = ocol_v[pl.ds(c0, _LANE)]
            for j in range(D):
                vals = plsc.load_gather(line_buf, [rows_in, offs + jnp.int32(j)])
                plsc.store_scatter(
                    out_buf, [rows_out, cols_base + jnp.int32(j)], vals
                )
            return carry

        lax.fori_loop(0, b_per_w // _LANE, grp, None, unroll=4)
        pltpu.sync_copy(out_buf, out_hbm.at[pl.ds(wid * l_per_w, l_per_w)])

    return emb


def kernel(x, table):
    B = x.shape[0]
    V, D = table.shape
    LINES = V * D // 128
    emb = _make_emb(B, V, D, LINES)
    out_lines = emb(x.astype(jnp.int32), table.reshape(LINES, 128))
    return out_lines.reshape(B, D)


# speedup vs baseline: 1.5659x; 1.5659x over previous
import functools

import jax
import jax.numpy as jnp
from jax import lax
from jax.experimental import pallas as pl
from jax.experimental.pallas import tpu as pltpu
from jax.experimental.pallas import tpu_sc as plsc

_BURST = 16


def _make_emb(B, V, D):
    info = plsc.get_sparse_core_info()
    nw = info.num_cores * info.num_subcores
    assert B % nw == 0
    b_per_w = B // nw
    n_bursts = b_per_w // _BURST

    mesh = plsc.VectorSubcoreMesh(core_axis_name="c", subcore_axis_name="s")

    @functools.partial(
        pl.kernel,
        mesh=mesh,
        out_type=jax.ShapeDtypeStruct((B, D), jnp.float32),
        scratch_types=[
            pltpu.VMEM((b_per_w,), jnp.int32),
            pltpu.VMEM((b_per_w, D), jnp.float32),
            pltpu.SemaphoreType.DMA,
        ],
        compiler_params=pltpu.CompilerParams(use_tc_tiling_on_sc=True),
    )
    def emb(idx_hbm, table_hbm, out_hbm, idx_v, rows_v, sem):
        wid = lax.axis_index("s") * info.num_cores + lax.axis_index("c")
        base = wid * b_per_w
        pltpu.sync_copy(idx_hbm.at[pl.ds(base, b_per_w)], idx_v)

        def burst(g, carry):
            c0 = g * _BURST
            v16 = idx_v[pl.ds(c0, _BURST)]
            for k in range(_BURST):
                s = v16[k]
                pltpu.async_copy(
                    table_hbm.at[pl.ds(s, 1)],
                    rows_v.at[pl.ds(c0 + k, 1)],
                    sem,
                )
            @pl.when(g > 0)
            def _drain():
                for k in range(_BURST):
                    pltpu.make_async_copy(
                        table_hbm.at[pl.ds(0, 1)],
                        rows_v.at[pl.ds(0, 1)],
                        sem,
                    ).wait()
            return carry

        lax.fori_loop(0, n_bursts, burst, None)
        for k in range(_BURST):
            pltpu.make_async_copy(
                table_hbm.at[pl.ds(0, 1)],
                rows_v.at[pl.ds(0, 1)],
                sem,
            ).wait()
        pltpu.sync_copy(rows_v, out_hbm.at[pl.ds(base, b_per_w)])

    return emb


def kernel(x, table):
    B = x.shape[0]
    V, D = table.shape
    emb = _make_emb(B, V, D)
    return emb(x.astype(jnp.int32), table)
